# baseline (device time: 65976 ns/iter reference)
import jax
import jax.numpy as jnp
from jax import lax
from jax.experimental import pallas as pl
from jax.experimental.pallas import tpu as pltpu

N_DEV = 8
B_PER = 2
SQ = 128
SKV = 128
HQ_PER = 4
DH = 64
D_MODEL = 512
QKD = HQ_PER * DH


def kernel(x, Wq, K_ext, V_ext, Wo):
    my = lax.axis_index("i")
    bf16 = jnp.bfloat16

    x2d = x.reshape(B_PER * SQ, D_MODEL).astype(bf16)
    wq = Wq.astype(bf16)
    wo = Wo.astype(bf16)

    def prep(ext):
        eb = lax.dynamic_slice_in_dim(ext, B_PER * my, B_PER, axis=0)
        eb = eb.astype(bf16).reshape(B_PER, SKV, N_DEV, HQ_PER, DH)
        idx = jnp.mod(my - jnp.arange(N_DEV), N_DEV)
        eb = jnp.take(eb, idx, axis=2)
        return jnp.transpose(eb, (2, 0, 3, 1, 4))

    k_arr = prep(K_ext)
    v_arr = prep(V_ext)

    def body(x_ref, wq_ref, wo_ref, k_ref, v_ref, out_ref,
             wq_comm, wo_comm, ctx_scratch,
             wq_send, wq_recv, wo_send, wo_recv):
        my_pos = lax.axis_index("i")
        left = lax.rem(my_pos + N_DEV - 1, N_DEV)
        right = lax.rem(my_pos + 1, N_DEV)

        barrier = pltpu.get_barrier_semaphore()
        for nbr in (left, right):
            pl.semaphore_signal(barrier, inc=1, device_id=(nbr,),
                                device_id_type=pl.DeviceIdType.MESH)
        pl.semaphore_wait(barrier, 2)

        x_val = x_ref[...]

        def contrib(h, wq_val, wo_val, is_first):
            q2 = lax.dot(x_val, wq_val,
                         preferred_element_type=jnp.float32).astype(bf16)
            for b in range(B_PER):
                for hh in range(HQ_PER):
                    q = q2[b * SQ:(b + 1) * SQ, hh * DH:(hh + 1) * DH]
                    k = k_ref[h, b, hh]
                    s = lax.dot_general(
                        q, k, (((1,), (1,)), ((), ())),
                        preferred_element_type=jnp.float32) * 0.125
                    m = jnp.max(s, axis=-1, keepdims=True)
                    w = jnp.exp(s - m)
                    w = w / jnp.sum(w, axis=-1, keepdims=True)
                    ctx = lax.dot(w.astype(bf16), v_ref[h, b, hh],
                                  preferred_element_type=jnp.float32)
                    ctx_scratch[b * SQ:(b + 1) * SQ,
                                hh * DH:(hh + 1) * DH] = ctx.astype(bf16)
            part = lax.dot(ctx_scratch[...], wo_val,
                           preferred_element_type=jnp.float32)
            if is_first:
                out_ref[...] = part
            else:
                out_ref[...] += part

        def mk(h):
            src_wq = wq_ref if h == 1 else wq_comm.at[h - 2]
            src_wo = wo_ref if h == 1 else wo_comm.at[h - 2]
            rd_wq = pltpu.make_async_remote_copy(
                src_ref=src_wq, dst_ref=wq_comm.at[h - 1],
                send_sem=wq_send.at[h - 1], recv_sem=wq_recv.at[h - 1],
                device_id=(right,), device_id_type=pl.DeviceIdType.MESH)
            rd_wo = pltpu.make_async_remote_copy(
                src_ref=src_wo, dst_ref=wo_comm.at[h - 1],
                send_sem=wo_send.at[h - 1], recv_sem=wo_recv.at[h - 1],
                device_id=(right,), device_id_type=pl.DeviceIdType.MESH)
            return rd_wq, rd_wo

        rds = {1: mk(1)}
        rds[1][0].start()
        rds[1][1].start()
        contrib(0, wq_ref[...], wo_ref[...], True)
        for h in range(1, N_DEV):
            rds[h][0].wait_recv()
            rds[h][1].wait_recv()
            if h < N_DEV - 1:
                rds[h + 1] = mk(h + 1)
                rds[h + 1][0].start()
                rds[h + 1][1].start()
            contrib(h, wq_comm[h - 1], wo_comm[h - 1], False)
        for h in range(1, N_DEV):
            rds[h][0].wait_send()
            rds[h][1].wait_send()

    out2d = pl.pallas_call(
        body,
        out_shape=jax.ShapeDtypeStruct((B_PER * SQ, D_MODEL), jnp.float32),
        in_specs=[pl.BlockSpec(memory_space=pltpu.VMEM)] * 5,
        out_specs=pl.BlockSpec(memory_space=pltpu.VMEM),
        scratch_shapes=[
            pltpu.VMEM((N_DEV - 1, D_MODEL, QKD), bf16),
            pltpu.VMEM((N_DEV - 1, QKD, D_MODEL), bf16),
            pltpu.VMEM((B_PER * SQ, QKD), bf16),
            pltpu.SemaphoreType.DMA((N_DEV - 1,)),
            pltpu.SemaphoreType.DMA((N_DEV - 1,)),
            pltpu.SemaphoreType.DMA((N_DEV - 1,)),
            pltpu.SemaphoreType.DMA((N_DEV - 1,)),
        ],
        compiler_params=pltpu.CompilerParams(collective_id=0),
    )(x2d, wq, wo, k_arr, v_arr)

    return out2d.reshape(B_PER, SQ, D_MODEL)


# device time: 43823 ns/iter; 1.5055x vs baseline; 1.5055x over previous
import jax
import jax.numpy as jnp
from jax import lax
from jax.experimental import pallas as pl
from jax.experimental.pallas import tpu as pltpu

N_DEV = 8
B_PER = 2
SQ = 128
SKV = 128
HQ_PER = 4
DH = 64
D_MODEL = 512
QKD = HQ_PER * DH

N_R = 4
N_L = 3
_OFFSETS = (0, -1, 1, -2, 2, -3, 3, -4)


def kernel(x, Wq, K_ext, V_ext, Wo):
    my = lax.axis_index("i")
    bf16 = jnp.bfloat16

    x2d = x.reshape(B_PER * SQ, D_MODEL).astype(bf16)
    wq = Wq.astype(bf16)
    wo = Wo.astype(bf16)

    def prep(ext):
        eb = lax.dynamic_slice_in_dim(ext, B_PER * my, B_PER, axis=0)
        eb = eb.astype(bf16).reshape(B_PER, SKV, N_DEV, HQ_PER, DH)
        idx = jnp.mod(my + jnp.asarray(_OFFSETS), N_DEV)
        eb = jnp.take(eb, idx, axis=2)
        return jnp.transpose(eb, (2, 0, 3, 1, 4))

    k_arr = prep(K_ext)
    v_arr = prep(V_ext)

    def body(x_ref, wq_ref, wo_ref, k_ref, v_ref, out_ref,
             rwq, rwo, lwq, lwo, ctx_scratch,
             r_send_wq, r_recv_wq, r_send_wo, r_recv_wo,
             l_send_wq, l_recv_wq, l_send_wo, l_recv_wo):
        my_pos = lax.axis_index("i")
        left = lax.rem(my_pos + N_DEV - 1, N_DEV)
        right = lax.rem(my_pos + 1, N_DEV)

        barrier = pltpu.get_barrier_semaphore()
        for nbr in (left, right):
            pl.semaphore_signal(barrier, inc=1, device_id=(nbr,),
                                device_id_type=pl.DeviceIdType.MESH)
        pl.semaphore_wait(barrier, 2)

        x_val = x_ref[...]

        def contrib(t, wq_val, wo_val, is_first=False):
            q2 = lax.dot(x_val, wq_val,
                         preferred_element_type=jnp.float32).astype(bf16)
            for b in range(B_PER):
                for hh in range(HQ_PER):
                    q = q2[b * SQ:(b + 1) * SQ, hh * DH:(hh + 1) * DH]
                    k = k_ref[t, b, hh]
                    s = lax.dot_general(
                        q, k, (((1,), (1,)), ((), ())),
                        preferred_element_type=jnp.float32) * 0.125
                    m = jnp.max(s, axis=-1, keepdims=True)
                    w = jnp.exp(s - m)
                    w = w / jnp.sum(w, axis=-1, keepdims=True)
                    ctx = lax.dot(w.astype(bf16), v_ref[t, b, hh],
                                  preferred_element_type=jnp.float32)
                    ctx_scratch[b * SQ:(b + 1) * SQ,
                                hh * DH:(hh + 1) * DH] = ctx.astype(bf16)
            part = lax.dot(ctx_scratch[...], wo_val,
                           preferred_element_type=jnp.float32)
            if is_first:
                out_ref[...] = part
            else:
                out_ref[...] += part

        def mk(src_wq, src_wo, wq_slot, wo_slot, sems, tgt):
            swq, rwq_s, swo, rwo_s = sems
            rd_wq = pltpu.make_async_remote_copy(
                src_ref=src_wq, dst_ref=wq_slot,
                send_sem=swq, recv_sem=rwq_s,
                device_id=(tgt,), device_id_type=pl.DeviceIdType.MESH)
            rd_wo = pltpu.make_async_remote_copy(
                src_ref=src_wo, dst_ref=wo_slot,
                send_sem=swo, recv_sem=rwo_s,
                device_id=(tgt,), device_id_type=pl.DeviceIdType.MESH)
            return rd_wq, rd_wo

        def mk_right(h):
            src_wq = wq_ref if h == 1 else rwq.at[h - 2]
            src_wo = wo_ref if h == 1 else rwo.at[h - 2]
            sems = (r_send_wq.at[h - 1], r_recv_wq.at[h - 1],
                    r_send_wo.at[h - 1], r_recv_wo.at[h - 1])
            return mk(src_wq, src_wo, rwq.at[h - 1], rwo.at[h - 1], sems,
                      right)

        def mk_left(h):
            src_wq = wq_ref if h == 1 else lwq.at[h - 2]
            src_wo = wo_ref if h == 1 else lwo.at[h - 2]
            sems = (l_send_wq.at[h - 1], l_recv_wq.at[h - 1],
                    l_send_wo.at[h - 1], l_recv_wo.at[h - 1])
            return mk(src_wq, src_wo, lwq.at[h - 1], lwo.at[h - 1], sems,
                      left)

        r = {1: mk_right(1)}
        l = {1: mk_left(1)}
        for rd in (*r[1], *l[1]):
            rd.start()
        contrib(0, wq_ref[...], wo_ref[...], is_first=True)

        for h in range(1, N_R + 1):
            r[h][0].wait_recv()
            r[h][1].wait_recv()
            if h < N_R:
                r[h + 1] = mk_right(h + 1)
                r[h + 1][0].start()
                r[h + 1][1].start()
            if h <= N_L:
                l[h][0].wait_recv()
                l[h][1].wait_recv()
                if h < N_L:
                    l[h + 1] = mk_left(h + 1)
                    l[h + 1][0].start()
                    l[h + 1][1].start()
            contrib(2 * h - 1, rwq[h - 1], rwo[h - 1])
            if h <= N_L:
                contrib(2 * h, lwq[h - 1], lwo[h - 1])

        for h in range(1, N_R + 1):
            r[h][0].wait_send()
            r[h][1].wait_send()
        for h in range(1, N_L + 1):
            l[h][0].wait_send()
            l[h][1].wait_send()

    out2d = pl.pallas_call(
        body,
        out_shape=jax.ShapeDtypeStruct((B_PER * SQ, D_MODEL), jnp.float32),
        in_specs=[pl.BlockSpec(memory_space=pltpu.VMEM)] * 5,
        out_specs=pl.BlockSpec(memory_space=pltpu.VMEM),
        scratch_shapes=[
            pltpu.VMEM((N_R, D_MODEL, QKD), bf16),
            pltpu.VMEM((N_R, QKD, D_MODEL), bf16),
            pltpu.VMEM((N_L, D_MODEL, QKD), bf16),
            pltpu.VMEM((N_L, QKD, D_MODEL), bf16),
            pltpu.VMEM((B_PER * SQ, QKD), bf16),
            pltpu.SemaphoreType.DMA((N_R,)),
            pltpu.SemaphoreType.DMA((N_R,)),
            pltpu.SemaphoreType.DMA((N_R,)),
            pltpu.SemaphoreType.DMA((N_R,)),
            pltpu.SemaphoreType.DMA((N_L,)),
            pltpu.SemaphoreType.DMA((N_L,)),
            pltpu.SemaphoreType.DMA((N_L,)),
            pltpu.SemaphoreType.DMA((N_L,)),
        ],
        compiler_params=pltpu.CompilerParams(collective_id=0),
    )(x2d, wq, wo, k_arr, v_arr)

    return out2d.reshape(B_PER, SQ, D_MODEL)


# device time: 42711 ns/iter; 1.5447x vs baseline; 1.0260x over previous
import jax
import jax.numpy as jnp
from jax import lax
from jax.experimental import pallas as pl
from jax.experimental.pallas import tpu as pltpu

N_DEV = 8
B_PER = 2
SQ = 128
SKV = 128
HQ_PER = 4
DH = 64
D_MODEL = 512
QKD = HQ_PER * DH

N_R = 4
N_L = 3
_OFFSETS = (0, -1, 1, -2, 2, -3, 3, -4)


def _rank(p):
    return jnp.where(p < 4, p, 11 - p)


def kernel(x, Wq, K_ext, V_ext, Wo):
    my = lax.axis_index("i")
    bf16 = jnp.bfloat16

    x2d = x.reshape(B_PER * SQ, D_MODEL).astype(bf16)
    wq = Wq.astype(bf16)
    wo = Wo.astype(bf16)

    def prep(ext):
        eb = lax.dynamic_slice_in_dim(ext, B_PER * my, B_PER, axis=0)
        eb = eb.astype(bf16).reshape(B_PER, SKV, N_DEV, HQ_PER, DH)
        idx = _rank(jnp.mod(_rank(my) + jnp.asarray(_OFFSETS), N_DEV))
        eb = jnp.take(eb, idx, axis=2)
        return jnp.transpose(eb, (2, 0, 3, 1, 4))

    k_arr = prep(K_ext)
    v_arr = prep(V_ext)

    def body(x_ref, wq_ref, wo_ref, k_ref, v_ref, out_ref,
             rwq, rwo, lwq, lwo, ctx_scratch,
             r_send_wq, r_recv_wq, r_send_wo, r_recv_wo,
             l_send_wq, l_recv_wq, l_send_wo, l_recv_wo):
        my_rank = _rank(lax.axis_index("i"))
        left = _rank(lax.rem(my_rank + N_DEV - 1, N_DEV))
        right = _rank(lax.rem(my_rank + 1, N_DEV))

        barrier = pltpu.get_barrier_semaphore()
        for nbr in (left, right):
            pl.semaphore_signal(barrier, inc=1, device_id=(nbr,),
                                device_id_type=pl.DeviceIdType.MESH)
        pl.semaphore_wait(barrier, 2)

        x_val = x_ref[...]

        def contrib(t, wq_val, wo_val, is_first=False):
            q2 = lax.dot(x_val, wq_val,
                         preferred_element_type=jnp.float32).astype(bf16)
            for b in range(B_PER):
                for hh in range(HQ_PER):
                    q = q2[b * SQ:(b + 1) * SQ, hh * DH:(hh + 1) * DH]
                    k = k_ref[t, b, hh]
                    s = lax.dot_general(
                        q, k, (((1,), (1,)), ((), ())),
                        preferred_element_type=jnp.float32) * 0.125
                    m = jnp.max(s, axis=-1, keepdims=True)
                    w = jnp.exp(s - m)
                    w = w / jnp.sum(w, axis=-1, keepdims=True)
                    ctx = lax.dot(w.astype(bf16), v_ref[t, b, hh],
                                  preferred_element_type=jnp.float32)
                    ctx_scratch[b * SQ:(b + 1) * SQ,
                                hh * DH:(hh + 1) * DH] = ctx.astype(bf16)
            part = lax.dot(ctx_scratch[...], wo_val,
                           preferred_element_type=jnp.float32)
            if is_first:
                out_ref[...] = part
            else:
                out_ref[...] += part

        def mk(src_wq, src_wo, wq_slot, wo_slot, sems, tgt):
            swq, rwq_s, swo, rwo_s = sems
            rd_wq = pltpu.make_async_remote_copy(
                src_ref=src_wq, dst_ref=wq_slot,
                send_sem=swq, recv_sem=rwq_s,
                device_id=(tgt,), device_id_type=pl.DeviceIdType.MESH)
            rd_wo = pltpu.make_async_remote_copy(
                src_ref=src_wo, dst_ref=wo_slot,
                send_sem=swo, recv_sem=rwo_s,
                device_id=(tgt,), device_id_type=pl.DeviceIdType.MESH)
            return rd_wq, rd_wo

        def mk_right(h):
            src_wq = wq_ref if h == 1 else rwq.at[h - 2]
            src_wo = wo_ref if h == 1 else rwo.at[h - 2]
            sems = (r_send_wq.at[h - 1], r_recv_wq.at[h - 1],
                    r_send_wo.at[h - 1], r_recv_wo.at[h - 1])
            return mk(src_wq, src_wo, rwq.at[h - 1], rwo.at[h - 1], sems,
                      right)

        def mk_left(h):
            src_wq = wq_ref if h == 1 else lwq.at[h - 2]
            src_wo = wo_ref if h == 1 else lwo.at[h - 2]
            sems = (l_send_wq.at[h - 1], l_recv_wq.at[h - 1],
                    l_send_wo.at[h - 1], l_recv_wo.at[h - 1])
            return mk(src_wq, src_wo, lwq.at[h - 1], lwo.at[h - 1], sems,
                      left)

        r = {1: mk_right(1)}
        l = {1: mk_left(1)}
        for rd in (*r[1], *l[1]):
            rd.start()
        contrib(0, wq_ref[...], wo_ref[...], is_first=True)

        for h in range(1, N_R + 1):
            r[h][0].wait_recv()
            r[h][1].wait_recv()
            if h < N_R:
                r[h + 1] = mk_right(h + 1)
                r[h + 1][0].start()
                r[h + 1][1].start()
            if h <= N_L:
                l[h][0].wait_recv()
                l[h][1].wait_recv()
                if h < N_L:
                    l[h + 1] = mk_left(h + 1)
                    l[h + 1][0].start()
                    l[h + 1][1].start()
            contrib(2 * h - 1, rwq[h - 1], rwo[h - 1])
            if h <= N_L:
                contrib(2 * h, lwq[h - 1], lwo[h - 1])

        for h in range(1, N_R + 1):
            r[h][0].wait_send()
            r[h][1].wait_send()
        for h in range(1, N_L + 1):
            l[h][0].wait_send()
            l[h][1].wait_send()

    out2d = pl.pallas_call(
        body,
        out_shape=jax.ShapeDtypeStruct((B_PER * SQ, D_MODEL), jnp.float32),
        in_specs=[pl.BlockSpec(memory_space=pltpu.VMEM)] * 5,
        out_specs=pl.BlockSpec(memory_space=pltpu.VMEM),
        scratch_shapes=[
            pltpu.VMEM((N_R, D_MODEL, QKD), bf16),
            pltpu.VMEM((N_R, QKD, D_MODEL), bf16),
            pltpu.VMEM((N_L, D_MODEL, QKD), bf16),
            pltpu.VMEM((N_L, QKD, D_MODEL), bf16),
            pltpu.VMEM((B_PER * SQ, QKD), bf16),
            pltpu.SemaphoreType.DMA((N_R,)),
            pltpu.SemaphoreType.DMA((N_R,)),
            pltpu.SemaphoreType.DMA((N_R,)),
            pltpu.SemaphoreType.DMA((N_R,)),
            pltpu.SemaphoreType.DMA((N_L,)),
            pltpu.SemaphoreType.DMA((N_L,)),
            pltpu.SemaphoreType.DMA((N_L,)),
            pltpu.SemaphoreType.DMA((N_L,)),
        ],
        compiler_params=pltpu.CompilerParams(collective_id=0),
    )(x2d, wq, wo, k_arr, v_arr)

    return out2d.reshape(B_PER, SQ, D_MODEL)
